# baseline (speedup 1.0000x reference)
import functools

import jax
import jax.numpy as jnp
from jax import lax
from jax.experimental import pallas as pl
from jax.experimental.pallas import tpu as pltpu
from jax.experimental.pallas import tpu_sc as plsc

_BATCH = 16384
_EMB = 32
_NTAB = 4
_VOCAB = 100000
_ROWW = 128
_VPR = _ROWW // _EMB
_PROWS = _VOCAB // _VPR
_NCHUNKS_PACK = _VOCAB // _ROWW
_TAIL0 = _NCHUNKS_PACK * _ROWW
_NC = 2
_NS = 16
_NW = _NC * _NS
_BPW = _BATCH // _NW
_CHUNK = 128
_NCHUNK = _BPW // _CHUNK

_mesh = plsc.VectorSubcoreMesh(core_axis_name="c", subcore_axis_name="s")


def _make_pack_kernel():
  p_ty = jax.ShapeDtypeStruct((_PROWS, _ROWW), jnp.float32)

  @functools.partial(
      pl.kernel,
      mesh=_mesh,
      out_type=(p_ty,) * _NTAB,
      compiler_params=pltpu.CompilerParams(needs_layout_passes=False),
      scratch_types=[
          pltpu.VMEM((_EMB, _ROWW), jnp.float32),
          pltpu.VMEM((_EMB, _ROWW), jnp.float32),
          pltpu.SemaphoreType.DMA,
      ],
  )
  def body(wgT, waT, woT, wzT, tails, pg, pa, po, pz, in_v, out_v, sem):
    wid = lax.axis_index("s") * _NC + lax.axis_index("c")
    slot = wid // _NTAB
    iota = lax.iota(jnp.int32, 16)
    tabs = (wgT, waT, woT, wzT)
    packs = (pg, pa, po, pz)
    for tt in range(_NTAB):

      @pl.when(wid % _NTAB == tt)
      def _(tt=tt):
        def do_chunk(m, _):
          k = slot + 8 * m

          @pl.when(k < _NCHUNKS_PACK)
          def _():
            pltpu.sync_copy(tabs[tt].at[:, pl.ds(k * _ROWW, _ROWW)], in_v)
            for r in range(_EMB):
              for j in range(_ROWW // 16):
                rows = iota + 16 * (j % 2)
                cols = jnp.full((16,), 4 * r + j // 2, jnp.int32)
                out_v[r, pl.ds(j * 16, 16)] = plsc.load_gather(
                    in_v, [rows, cols])
            pltpu.sync_copy(out_v, packs[tt].at[pl.ds(k * _EMB, _EMB)])
          return ()

        lax.fori_loop(0, 98, do_chunk, ())

      @pl.when(wid == _NW - 1 - tt)
      def _(tt=tt):
        pltpu.sync_copy(tails.at[tt], in_v.at[pl.ds(0, 8)])
        pltpu.sync_copy(in_v.at[pl.ds(0, 8)],
                        packs[tt].at[pl.ds(_TAIL0 // _VPR, 8)])

  return body


def _make_gather_kernel():
  @functools.partial(
      pl.kernel,
      mesh=_mesh,
      out_type=jax.ShapeDtypeStruct((_BATCH, _NTAB * _EMB), jnp.float32),
      scratch_types=[
          pltpu.VMEM((_NTAB, _CHUNK), jnp.int32),
          pltpu.VMEM((_NTAB, _CHUNK), jnp.int32),
          pltpu.VMEM((_NTAB, _CHUNK, _ROWW), jnp.float32),
          pltpu.VMEM((_CHUNK, _NTAB * _EMB), jnp.float32),
          pltpu.SemaphoreType.DMA,
      ],
  )
  def body(xT, pg, pa, po, pz, out_hbm, idx_v, q_v, rows_v, out_v, gsem):
    wid = lax.axis_index("s") * _NC + lax.axis_index("c")
    base = wid * _BPW
    tables = (pg, pa, po, pz)
    for j in range(_NCHUNK):
      b0 = base + j * _CHUNK
      pltpu.sync_copy(xT.at[:, pl.ds(b0, _CHUNK)], idx_v)
      for t in range(_NTAB):
        for v in range(_CHUNK // 16):
          q_v[t, pl.ds(v * 16, 16)] = jax.lax.shift_right_logical(
              idx_v[t, pl.ds(v * 16, 16)], 2)
      copies = [
          pltpu.async_copy(tables[t].at[q_v.at[t]], rows_v.at[t], gsem)
          for t in range(_NTAB)
      ]
      for cp in copies:
        cp.wait()

      def select(g, _):
        for t in range(_NTAB):
          iv = idx_v[t, pl.ds(g * 16, 16)]
          for l in range(16):
            off = (iv[l] & (_VPR - 1)) * _EMB
            b = g * 16 + l
            for k in range(_EMB // 16):
              out_v[b, pl.ds(t * _EMB + k * 16, 16)] = (
                  rows_v[t, b, pl.ds(off + k * 16, 16)])
        return ()

      lax.fori_loop(0, _CHUNK // 16, select, ())
      pltpu.sync_copy(out_v, out_hbm.at[pl.ds(b0, _CHUNK)])

  return body


_pack = _make_pack_kernel()
_gather = _make_gather_kernel()


def kernel(x, W_gender, W_age, W_occupation, W_zip):
  ws = (W_gender, W_age, W_occupation, W_zip)
  tails = jnp.stack([w[_TAIL0:].reshape(8, _ROWW) for w in ws])
  packed = _pack(*[w.T for w in ws], tails)
  return _gather(x.T, *packed)


# speedup vs baseline: 1.1857x; 1.1857x over previous
import functools

import jax
import jax.numpy as jnp
from jax import lax
from jax.experimental import pallas as pl
from jax.experimental.pallas import tpu as pltpu
from jax.experimental.pallas import tpu_sc as plsc

_BATCH = 16384
_EMB = 32
_NTAB = 4
_VOCAB = 100000
_ROWW = 128
_VPR = _ROWW // _EMB
_PROWS = _VOCAB // _VPR
_NCHUNKS_PACK = _VOCAB // _ROWW
_TAIL0 = _NCHUNKS_PACK * _ROWW
_NC = 2
_NS = 16
_NW = _NC * _NS
_BPW = _BATCH // _NW
_CHUNK = 128
_NCHUNK = _BPW // _CHUNK

_mesh = plsc.VectorSubcoreMesh(core_axis_name="c", subcore_axis_name="s")


def _make_pack_kernel():
  p_ty = jax.ShapeDtypeStruct((_PROWS, _ROWW), jnp.float32)

  @functools.partial(
      pl.kernel,
      mesh=_mesh,
      out_type=(p_ty,) * _NTAB,
      compiler_params=pltpu.CompilerParams(needs_layout_passes=False),
      scratch_types=[
          pltpu.VMEM((3, _EMB, _ROWW), jnp.float32),
          pltpu.VMEM((2, _EMB, _ROWW), jnp.float32),
          pltpu.SemaphoreType.DMA,
          pltpu.SemaphoreType.DMA,
      ],
  )
  def body(wgT, waT, woT, wzT, tails, pg, pa, po, pz, in_v, out_v,
           sin, sout):
    wid = lax.axis_index("s") * _NC + lax.axis_index("c")
    slot = wid // _NTAB
    iota = lax.iota(jnp.int32, 16)
    tabs = (wgT, waT, woT, wzT)
    packs = (pg, pa, po, pz)
    _MAIN = 97

    def transpose_chunk(src, dst):
      for r in range(_EMB):
        for j in range(_ROWW // 16):
          rows = iota + 16 * (j % 2)
          cols = jnp.full((16,), 4 * r + j // 2, jnp.int32)
          dst[r, pl.ds(j * 16, 16)] = plsc.load_gather(src, [rows, cols])

    for tt in range(_NTAB):

      @pl.when(wid % _NTAB == tt)
      def _(tt=tt):
        def in_copy(m, buf):
          k = slot + 8 * m
          return pltpu.make_async_copy(
              tabs[tt].at[:, pl.ds(k * _ROWW, _ROWW)], in_v.at[buf], sin)

        def out_copy(m, buf):
          k = slot + 8 * m
          return pltpu.make_async_copy(
              out_v.at[buf], packs[tt].at[pl.ds(k * _EMB, _EMB)], sout)

        for p in range(3):
          in_copy(p, p).start()

        def step(m, _):
          in_copy(m, m % 3).wait()

          @pl.when(m >= 2)
          def _():
            out_copy(m - 2, m % 2).wait()

          transpose_chunk(in_v.at[m % 3], out_v.at[m % 2])
          out_copy(m, m % 2).start()

          @pl.when(m + 3 < _MAIN)
          def _():
            in_copy(m + 3, (m + 3) % 3).start()
          return ()

        lax.fori_loop(0, _MAIN, step, ())
        for m in (_MAIN - 2, _MAIN - 1):
          out_copy(m, m % 2).wait()

        @pl.when(slot < _NCHUNKS_PACK - 8 * _MAIN)
        def _():
          in_copy(_MAIN, 0).start()
          in_copy(_MAIN, 0).wait()
          transpose_chunk(in_v.at[0], out_v.at[0])
          out_copy(_MAIN, 0).start()
          out_copy(_MAIN, 0).wait()

      @pl.when(wid == _NW - 1 - tt)
      def _(tt=tt):
        pltpu.sync_copy(tails.at[tt], in_v.at[0, pl.ds(0, 8)])
        pltpu.sync_copy(in_v.at[0, pl.ds(0, 8)],
                        packs[tt].at[pl.ds(_TAIL0 // _VPR, 8)])

  return body


def _make_gather_kernel():
  @functools.partial(
      pl.kernel,
      mesh=_mesh,
      out_type=jax.ShapeDtypeStruct((_BATCH, _NTAB * _EMB), jnp.float32),
      scratch_types=[
          pltpu.VMEM((_NTAB, _CHUNK), jnp.int32),
          pltpu.VMEM((_NTAB, _CHUNK), jnp.int32),
          pltpu.VMEM((_NTAB, _CHUNK, _ROWW), jnp.float32),
          pltpu.VMEM((_CHUNK, _NTAB * _EMB), jnp.float32),
          pltpu.SemaphoreType.DMA,
      ],
  )
  def body(xT, pg, pa, po, pz, out_hbm, idx_v, q_v, rows_v, out_v, gsem):
    wid = lax.axis_index("s") * _NC + lax.axis_index("c")
    base = wid * _BPW
    tables = (pg, pa, po, pz)
    for j in range(_NCHUNK):
      b0 = base + j * _CHUNK
      pltpu.sync_copy(xT.at[:, pl.ds(b0, _CHUNK)], idx_v)
      for t in range(_NTAB):
        for v in range(_CHUNK // 16):
          q_v[t, pl.ds(v * 16, 16)] = jax.lax.shift_right_logical(
              idx_v[t, pl.ds(v * 16, 16)], 2)
      copies = [
          pltpu.async_copy(tables[t].at[q_v.at[t]], rows_v.at[t], gsem)
          for t in range(_NTAB)
      ]
      for cp in copies:
        cp.wait()

      def select(g, _):
        for t in range(_NTAB):
          iv = idx_v[t, pl.ds(g * 16, 16)]
          for l in range(16):
            off = (iv[l] & (_VPR - 1)) * _EMB
            b = g * 16 + l
            for k in range(_EMB // 16):
              out_v[b, pl.ds(t * _EMB + k * 16, 16)] = (
                  rows_v[t, b, pl.ds(off + k * 16, 16)])
        return ()

      lax.fori_loop(0, _CHUNK // 16, select, ())
      pltpu.sync_copy(out_v, out_hbm.at[pl.ds(b0, _CHUNK)])

  return body


_pack = _make_pack_kernel()
_gather = _make_gather_kernel()


def kernel(x, W_gender, W_age, W_occupation, W_zip):
  ws = (W_gender, W_age, W_occupation, W_zip)
  tails = jnp.stack([w[_TAIL0:].reshape(8, _ROWW) for w in ws])
  packed = _pack(*[w.T for w in ws], tails)
  return _gather(x.T, *packed)


# speedup vs baseline: 1.4815x; 1.2495x over previous
import functools

import jax
import jax.numpy as jnp
from jax import lax
from jax.experimental import pallas as pl
from jax.experimental.pallas import tpu as pltpu
from jax.experimental.pallas import tpu_sc as plsc

_BATCH = 16384
_EMB = 32
_NTAB = 4
_VOCAB = 100000
_ROWW = 128
_VPR = _ROWW // _EMB
_PROWS = _VOCAB // _VPR
_NCHUNKS_PACK = _VOCAB // _ROWW
_TAIL0 = _NCHUNKS_PACK * _ROWW
_NC = 2
_NS = 16
_NW = _NC * _NS
_BPW = _BATCH // _NW
_CHUNK = 128
_NCHUNK = _BPW // _CHUNK

_mesh = plsc.VectorSubcoreMesh(core_axis_name="c", subcore_axis_name="s")


def _make_pack_kernel():
  p_ty = jax.ShapeDtypeStruct((_PROWS, _ROWW), jnp.float32)

  @functools.partial(
      pl.kernel,
      mesh=_mesh,
      out_type=(p_ty,) * _NTAB,
      compiler_params=pltpu.CompilerParams(needs_layout_passes=False),
      scratch_types=[
          pltpu.VMEM((3, _EMB, _ROWW), jnp.float32),
          pltpu.VMEM((2, _EMB, _ROWW), jnp.float32),
          pltpu.SemaphoreType.DMA,
          pltpu.SemaphoreType.DMA,
      ],
  )
  def body(wgT, waT, woT, wzT, tails, pg, pa, po, pz, in_v, out_v,
           sin, sout):
    wid = lax.axis_index("s") * _NC + lax.axis_index("c")
    slot = wid // _NTAB
    iota = lax.iota(jnp.int32, 16)
    tabs = (wgT, waT, woT, wzT)
    packs = (pg, pa, po, pz)
    _MAIN = 97

    rows8 = [(iota + 16 * j) >> 2 for j in range(_ROWW // 16)]
    colb8 = [((iota + 16 * j) & 3) * _EMB for j in range(_ROWW // 16)]

    def transpose_chunk(src, dst):
      for e in range(_EMB):
        for j in range(_ROWW // 16):
          v = src[e, pl.ds(j * 16, 16)]
          plsc.store_scatter(dst, [rows8[j], colb8[j] + e], v)

    for tt in range(_NTAB):

      @pl.when(wid % _NTAB == tt)
      def _(tt=tt):
        def in_copy(m, buf):
          k = slot + 8 * m
          return pltpu.make_async_copy(
              tabs[tt].at[:, pl.ds(k * _ROWW, _ROWW)], in_v.at[buf], sin)

        def out_copy(m, buf):
          k = slot + 8 * m
          return pltpu.make_async_copy(
              out_v.at[buf], packs[tt].at[pl.ds(k * _EMB, _EMB)], sout)

        for p in range(3):
          in_copy(p, p).start()

        def step(m, _):
          in_copy(m, m % 3).wait()

          @pl.when(m >= 2)
          def _():
            out_copy(m - 2, m % 2).wait()

          transpose_chunk(in_v.at[m % 3], out_v.at[m % 2])
          out_copy(m, m % 2).start()

          @pl.when(m + 3 < _MAIN)
          def _():
            in_copy(m + 3, (m + 3) % 3).start()
          return ()

        lax.fori_loop(0, _MAIN, step, ())
        for m in (_MAIN - 2, _MAIN - 1):
          out_copy(m, m % 2).wait()

        @pl.when(slot < _NCHUNKS_PACK - 8 * _MAIN)
        def _():
          in_copy(_MAIN, 0).start()
          in_copy(_MAIN, 0).wait()
          transpose_chunk(in_v.at[0], out_v.at[0])
          out_copy(_MAIN, 0).start()
          out_copy(_MAIN, 0).wait()

      @pl.when(wid == _NW - 1 - tt)
      def _(tt=tt):
        pltpu.sync_copy(tails.at[tt], in_v.at[0, pl.ds(0, 8)])
        pltpu.sync_copy(in_v.at[0, pl.ds(0, 8)],
                        packs[tt].at[pl.ds(_TAIL0 // _VPR, 8)])

  return body


def _make_gather_kernel():
  @functools.partial(
      pl.kernel,
      mesh=_mesh,
      out_type=jax.ShapeDtypeStruct((_BATCH, _NTAB * _EMB), jnp.float32),
      scratch_types=[
          pltpu.VMEM((_NTAB, _CHUNK), jnp.int32),
          pltpu.VMEM((_NTAB, _CHUNK), jnp.int32),
          pltpu.VMEM((_NTAB, _CHUNK, _ROWW), jnp.float32),
          pltpu.VMEM((_CHUNK, _NTAB * _EMB), jnp.float32),
          pltpu.SemaphoreType.DMA,
      ],
  )
  def body(xT, pg, pa, po, pz, out_hbm, idx_v, q_v, rows_v, out_v, gsem):
    wid = lax.axis_index("s") * _NC + lax.axis_index("c")
    base = wid * _BPW
    tables = (pg, pa, po, pz)
    for j in range(_NCHUNK):
      b0 = base + j * _CHUNK
      pltpu.sync_copy(xT.at[:, pl.ds(b0, _CHUNK)], idx_v)
      for t in range(_NTAB):
        for v in range(_CHUNK // 16):
          q_v[t, pl.ds(v * 16, 16)] = jax.lax.shift_right_logical(
              idx_v[t, pl.ds(v * 16, 16)], 2)
      copies = [
          pltpu.async_copy(tables[t].at[q_v.at[t]], rows_v.at[t], gsem)
          for t in range(_NTAB)
      ]
      for cp in copies:
        cp.wait()

      def select(g, _):
        for t in range(_NTAB):
          iv = idx_v[t, pl.ds(g * 16, 16)]
          for l in range(16):
            off = (iv[l] & (_VPR - 1)) * _EMB
            b = g * 16 + l
            for k in range(_EMB // 16):
              out_v[b, pl.ds(t * _EMB + k * 16, 16)] = (
                  rows_v[t, b, pl.ds(off + k * 16, 16)])
        return ()

      lax.fori_loop(0, _CHUNK // 16, select, ())
      pltpu.sync_copy(out_v, out_hbm.at[pl.ds(b0, _CHUNK)])

  return body


_pack = _make_pack_kernel()
_gather = _make_gather_kernel()


def kernel(x, W_gender, W_age, W_occupation, W_zip):
  ws = (W_gender, W_age, W_occupation, W_zip)
  tails = jnp.stack([w[_TAIL0:].reshape(8, _ROWW) for w in ws])
  packed = _pack(*[w.T for w in ws], tails)
  return _gather(x.T, *packed)
